# baseline (speedup 1.0000x reference)
import jax
import jax.numpy as jnp
from jax import lax
from jax.experimental import pallas as pl
from jax.experimental.pallas import tpu as pltpu
from jax.experimental.pallas import tpu_sc as plsc

_E = 64
_K = 2
_TILE = 8192

_NC = 2
_NS = 16
_NW = _NC * _NS
_CHUNK = 256


def _probs_kernel(x_ref, w_ref, probs_ref):
    x = x_ref[...]
    w = w_ref[...]
    logits = jax.lax.dot_general(
        x, w, (((1,), (1,)), ((), ())), preferred_element_type=jnp.float32
    )
    m = jnp.max(logits, axis=-1, keepdims=True)
    e = jnp.exp(logits - m)
    s = jnp.sum(e, axis=-1, keepdims=True)
    probs_ref[...] = e * (1.0 / s)


def _tc_probs(x_flat, W):
    tokens, d = x_flat.shape
    return pl.pallas_call(
        _probs_kernel,
        grid=(tokens // _TILE,),
        in_specs=[
            pl.BlockSpec((_TILE, d), lambda i: (i, 0)),
            pl.BlockSpec((_E, d), lambda i: (0, 0)),
        ],
        out_specs=pl.BlockSpec((_TILE, _E), lambda i: (i, 0)),
        out_shape=jax.ShapeDtypeStruct((tokens, _E), jnp.float32),
        compiler_params=pltpu.CompilerParams(
            dimension_semantics=("parallel",),
        ),
    )(x_flat, W)


def _sc_topk_body(probs_hbm, vals_hbm, idx_hbm, pbuf0, pbuf1, vbuf0, vbuf1,
                  ibuf0, ibuf1, sem0, sem1, osem0, osem1):
    tokens = probs_hbm.shape[0]
    tw = tokens // _NW
    wid = lax.axis_index("s") * _NC + lax.axis_index("c")
    base = wid * tw
    lane = lax.iota(jnp.int32, 16)
    neg = jnp.full((16,), jnp.int32(-(2**31)), jnp.int32)
    n_chunks = tw // _CHUNK
    pbufs = (pbuf0, pbuf1)
    sems = (sem0, sem1)
    vbufs = (vbuf0, vbuf1)
    ibufs = (ibuf0, ibuf1)
    osems = (osem0, osem1)

    def start_dma(c):
        start = base + c * _CHUNK
        return pltpu.async_copy(
            probs_hbm.at[pl.ds(start, _CHUNK)],
            pbufs[c % 2],
            sems[c % 2],
        )

    cur = start_dma(0)
    outcps = [None, None]
    for c in range(n_chunks):
        nxt = start_dma(c + 1) if c + 1 < n_chunks else None
        cur.wait()
        pbuf = pbufs[c % 2]
        vbuf = vbufs[c % 2]
        ibuf = ibufs[c % 2]
        if outcps[c % 2] is not None:
            for cp in outcps[c % 2]:
                cp.wait()

        @plsc.parallel_loop(0, _CHUNK // 16, unroll=2)
        def group_body(g):
            rows = lane + g * 16
            m1v = neg
            m1i = neg
            m2v = neg
            m2i = neg
            for i in range(_E):
                evec = (lane + i) & (_E - 1)
                p = plsc.bitcast(plsc.load_gather(pbuf, [rows, evec]), jnp.int32)
                c1 = p > m1v
                t = jnp.where(c1, m1v, p)
                ti = jnp.where(c1, m1i, evec)
                c2 = t > m2v
                m2v = jnp.where(c2, t, m2v)
                m2i = jnp.where(c2, ti, m2i)
                m1v = jnp.where(c1, p, m1v)
                m1i = jnp.where(c1, evec, m1i)
            obase = rows * _K
            plsc.store_scatter(vbuf, [obase], plsc.bitcast(m1v, jnp.float32))
            plsc.store_scatter(vbuf, [obase + 1], plsc.bitcast(m2v, jnp.float32))
            plsc.store_scatter(ibuf, [obase], m1i)
            plsc.store_scatter(ibuf, [obase + 1], m2i)

        start = base + c * _CHUNK
        outcps[c % 2] = (
            pltpu.async_copy(
                vbuf, vals_hbm.at[pl.ds(start * _K, _CHUNK * _K)], osems[c % 2]
            ),
            pltpu.async_copy(
                ibuf, idx_hbm.at[pl.ds(start * _K, _CHUNK * _K)], osems[c % 2]
            ),
        )
        cur = nxt
    for cps in outcps:
        if cps is not None:
            for cp in cps:
                cp.wait()


def _sc_topk(probs):
    tokens = probs.shape[0]
    vals, idx = pl.kernel(
        _sc_topk_body,
        out_type=[
            jax.ShapeDtypeStruct((tokens * _K,), jnp.float32),
            jax.ShapeDtypeStruct((tokens * _K,), jnp.int32),
        ],
        mesh=plsc.VectorSubcoreMesh(core_axis_name="c", subcore_axis_name="s"),
        compiler_params=pltpu.CompilerParams(
            needs_layout_passes=False, use_tc_tiling_on_sc=True
        ),
        scratch_types=[
            pltpu.VMEM((_CHUNK, _E), jnp.float32),
            pltpu.VMEM((_CHUNK, _E), jnp.float32),
            pltpu.VMEM((_CHUNK * _K,), jnp.float32),
            pltpu.VMEM((_CHUNK * _K,), jnp.float32),
            pltpu.VMEM((_CHUNK * _K,), jnp.int32),
            pltpu.VMEM((_CHUNK * _K,), jnp.int32),
            pltpu.SemaphoreType.DMA,
            pltpu.SemaphoreType.DMA,
            pltpu.SemaphoreType.DMA,
            pltpu.SemaphoreType.DMA,
        ],
    )(probs)
    return vals.reshape(tokens, _K), idx.reshape(tokens, _K)


def kernel(x, W):
    b, n, d = x.shape
    x_flat = x.reshape(b * n, d)
    probs = _tc_probs(x_flat, W)
    vals, idx = _sc_topk(probs)
    return (probs, vals, idx)


# speedup vs baseline: 1.0035x; 1.0035x over previous
import jax
import jax.numpy as jnp
from jax import lax
from jax.experimental import pallas as pl
from jax.experimental.pallas import tpu as pltpu
from jax.experimental.pallas import tpu_sc as plsc

_E = 64
_K = 2
_TILE = 4096

_NC = 2
_NS = 16
_NW = _NC * _NS
_CHUNK = 128


def _probs_kernel(x_ref, w_ref, probs_ref):
    x = x_ref[...]
    w = w_ref[...]
    logits = jax.lax.dot_general(
        x, w, (((1,), (1,)), ((), ())), preferred_element_type=jnp.float32
    )
    m = jnp.max(logits, axis=-1, keepdims=True)
    e = jnp.exp(logits - m)
    s = jnp.sum(e, axis=-1, keepdims=True)
    probs_ref[...] = e * (1.0 / s)


def _tc_probs(x_flat, W):
    tokens, d = x_flat.shape
    return pl.pallas_call(
        _probs_kernel,
        grid=(tokens // _TILE,),
        in_specs=[
            pl.BlockSpec((_TILE, d), lambda i: (i, 0)),
            pl.BlockSpec((_E, d), lambda i: (0, 0)),
        ],
        out_specs=pl.BlockSpec((_TILE, _E), lambda i: (i, 0)),
        out_shape=jax.ShapeDtypeStruct((tokens, _E), jnp.float32),
        compiler_params=pltpu.CompilerParams(
            dimension_semantics=("parallel",),
        ),
    )(x_flat, W)


def _sc_topk_body(probs_hbm, vals_hbm, idx_hbm, pbuf0, pbuf1, vbuf0, vbuf1,
                  ibuf0, ibuf1, sem0, sem1, osem0, osem1):
    tokens = probs_hbm.shape[0]
    tw = tokens // _NW
    wid = lax.axis_index("s") * _NC + lax.axis_index("c")
    base = wid * tw
    lane = lax.iota(jnp.int32, 16)
    neg = jnp.full((16,), jnp.int32(-(2**31)), jnp.int32)
    n_chunks = tw // _CHUNK
    pbufs = (pbuf0, pbuf1)
    sems = (sem0, sem1)
    vbufs = (vbuf0, vbuf1)
    ibufs = (ibuf0, ibuf1)
    osems = (osem0, osem1)

    def start_dma(c):
        start = base + c * _CHUNK
        return pltpu.async_copy(
            probs_hbm.at[pl.ds(start, _CHUNK)],
            pbufs[c % 2],
            sems[c % 2],
        )

    cur = start_dma(0)
    outcps = [None, None]
    for c in range(n_chunks):
        nxt = start_dma(c + 1) if c + 1 < n_chunks else None
        cur.wait()
        pbuf = pbufs[c % 2]
        vbuf = vbufs[c % 2]
        ibuf = ibufs[c % 2]
        if outcps[c % 2] is not None:
            for cp in outcps[c % 2]:
                cp.wait()

        @plsc.parallel_loop(0, _CHUNK // 16, unroll=2)
        def group_body(g):
            rows = lane + g * 16
            m1v = neg
            m1i = neg
            m2v = neg
            m2i = neg
            for i in range(_E):
                evec = (lane + i) & (_E - 1)
                p = plsc.bitcast(plsc.load_gather(pbuf, [rows, evec]), jnp.int32)
                c1 = p > m1v
                t = jnp.where(c1, m1v, p)
                ti = jnp.where(c1, m1i, evec)
                c2 = t > m2v
                m2v = jnp.where(c2, t, m2v)
                m2i = jnp.where(c2, ti, m2i)
                m1v = jnp.where(c1, p, m1v)
                m1i = jnp.where(c1, evec, m1i)
            obase = rows * _K
            plsc.store_scatter(vbuf, [obase], plsc.bitcast(m1v, jnp.float32))
            plsc.store_scatter(vbuf, [obase + 1], plsc.bitcast(m2v, jnp.float32))
            plsc.store_scatter(ibuf, [obase], m1i)
            plsc.store_scatter(ibuf, [obase + 1], m2i)

        start = base + c * _CHUNK
        outcps[c % 2] = (
            pltpu.async_copy(
                vbuf, vals_hbm.at[pl.ds(start * _K, _CHUNK * _K)], osems[c % 2]
            ),
            pltpu.async_copy(
                ibuf, idx_hbm.at[pl.ds(start * _K, _CHUNK * _K)], osems[c % 2]
            ),
        )
        cur = nxt
    for cps in outcps:
        if cps is not None:
            for cp in cps:
                cp.wait()


def _sc_topk(probs):
    tokens = probs.shape[0]
    vals, idx = pl.kernel(
        _sc_topk_body,
        out_type=[
            jax.ShapeDtypeStruct((tokens * _K,), jnp.float32),
            jax.ShapeDtypeStruct((tokens * _K,), jnp.int32),
        ],
        mesh=plsc.VectorSubcoreMesh(core_axis_name="c", subcore_axis_name="s"),
        compiler_params=pltpu.CompilerParams(
            needs_layout_passes=False, use_tc_tiling_on_sc=True
        ),
        scratch_types=[
            pltpu.VMEM((_CHUNK, _E), jnp.float32),
            pltpu.VMEM((_CHUNK, _E), jnp.float32),
            pltpu.VMEM((_CHUNK * _K,), jnp.float32),
            pltpu.VMEM((_CHUNK * _K,), jnp.float32),
            pltpu.VMEM((_CHUNK * _K,), jnp.int32),
            pltpu.VMEM((_CHUNK * _K,), jnp.int32),
            pltpu.SemaphoreType.DMA,
            pltpu.SemaphoreType.DMA,
            pltpu.SemaphoreType.DMA,
            pltpu.SemaphoreType.DMA,
        ],
    )(probs)
    return vals.reshape(tokens, _K), idx.reshape(tokens, _K)


def kernel(x, W):
    b, n, d = x.shape
    x_flat = x.reshape(b * n, d)
    probs = _tc_probs(x_flat, W)
    vals, idx = _sc_topk(probs)
    return (probs, vals, idx)


# speedup vs baseline: 1.0163x; 1.0128x over previous
import jax
import jax.numpy as jnp
from jax import lax
from jax.experimental import pallas as pl
from jax.experimental.pallas import tpu as pltpu
from jax.experimental.pallas import tpu_sc as plsc

_E = 64
_K = 2
_TILE = 4096

_NC = 2
_NS = 16
_NW = _NC * _NS
_CHUNK = 256


def _probs_kernel(x_ref, w_ref, probs_ref):
    x = x_ref[...]
    w = w_ref[...]
    logits = jax.lax.dot_general(
        x, w, (((1,), (1,)), ((), ())), preferred_element_type=jnp.float32
    )
    m = jnp.max(logits, axis=-1, keepdims=True)
    e = jnp.exp(logits - m)
    s = jnp.sum(e, axis=-1, keepdims=True)
    probs_ref[...] = e * (1.0 / s)


def _tc_probs(x_flat, W):
    tokens, d = x_flat.shape
    return pl.pallas_call(
        _probs_kernel,
        grid=(tokens // _TILE,),
        in_specs=[
            pl.BlockSpec((_TILE, d), lambda i: (i, 0)),
            pl.BlockSpec((_E, d), lambda i: (0, 0)),
        ],
        out_specs=pl.BlockSpec((_TILE, _E), lambda i: (i, 0)),
        out_shape=jax.ShapeDtypeStruct((tokens, _E), jnp.float32),
        compiler_params=pltpu.CompilerParams(
            dimension_semantics=("parallel",),
        ),
    )(x_flat, W)


def _sc_topk_body(probs_hbm, vals_hbm, idx_hbm, pbuf0, pbuf1, vbuf0, vbuf1,
                  ibuf0, ibuf1, sem0, sem1, osem0, osem1):
    tokens = probs_hbm.shape[0]
    tw = tokens // _NW
    wid = lax.axis_index("s") * _NC + lax.axis_index("c")
    base = wid * tw
    lane = lax.iota(jnp.int32, 16)
    neg = jnp.full((16,), jnp.int32(-(2**31)), jnp.int32)
    n_chunks = tw // _CHUNK
    pbufs = (pbuf0, pbuf1)
    sems = (sem0, sem1)
    vbufs = (vbuf0, vbuf1)
    ibufs = (ibuf0, ibuf1)
    osems = (osem0, osem1)

    def start_dma(c):
        start = base + c * _CHUNK
        return pltpu.async_copy(
            probs_hbm.at[pl.ds(start, _CHUNK)],
            pbufs[c % 2],
            sems[c % 2],
        )

    cur = start_dma(0)
    outcps = [None, None]
    for c in range(n_chunks):
        nxt = start_dma(c + 1) if c + 1 < n_chunks else None
        cur.wait()
        pbuf = pbufs[c % 2]
        vbuf = vbufs[c % 2]
        ibuf = ibufs[c % 2]
        if outcps[c % 2] is not None:
            for cp in outcps[c % 2]:
                cp.wait()

        @plsc.parallel_loop(0, _CHUNK // 16, unroll=2)
        def group_body(g):
            rows = lane + g * 16
            m1v = neg
            m1i = neg
            m2v = neg
            m2i = neg
            for i in range(_E):
                evec = (lane + i) & (_E - 1)
                p = plsc.bitcast(plsc.load_gather(pbuf, [rows, evec]), jnp.int32)
                c1 = p > m1v
                t = jnp.where(c1, m1v, p)
                ti = jnp.where(c1, m1i, evec)
                c2 = t > m2v
                m2v = jnp.where(c2, t, m2v)
                m2i = jnp.where(c2, ti, m2i)
                m1v = jnp.where(c1, p, m1v)
                m1i = jnp.where(c1, evec, m1i)
            obase = rows * _K
            plsc.store_scatter(vbuf, [obase], plsc.bitcast(m1v, jnp.float32))
            plsc.store_scatter(vbuf, [obase + 1], plsc.bitcast(m2v, jnp.float32))
            plsc.store_scatter(ibuf, [obase], m1i)
            plsc.store_scatter(ibuf, [obase + 1], m2i)

        start = base + c * _CHUNK
        outcps[c % 2] = (
            pltpu.async_copy(
                vbuf, vals_hbm.at[pl.ds(start * _K, _CHUNK * _K)], osems[c % 2]
            ),
            pltpu.async_copy(
                ibuf, idx_hbm.at[pl.ds(start * _K, _CHUNK * _K)], osems[c % 2]
            ),
        )
        cur = nxt
    for cps in outcps:
        if cps is not None:
            for cp in cps:
                cp.wait()


def _sc_topk(probs):
    tokens = probs.shape[0]
    vals, idx = pl.kernel(
        _sc_topk_body,
        out_type=[
            jax.ShapeDtypeStruct((tokens * _K,), jnp.float32),
            jax.ShapeDtypeStruct((tokens * _K,), jnp.int32),
        ],
        mesh=plsc.VectorSubcoreMesh(core_axis_name="c", subcore_axis_name="s"),
        compiler_params=pltpu.CompilerParams(
            needs_layout_passes=False, use_tc_tiling_on_sc=True
        ),
        scratch_types=[
            pltpu.VMEM((_CHUNK, _E), jnp.float32),
            pltpu.VMEM((_CHUNK, _E), jnp.float32),
            pltpu.VMEM((_CHUNK * _K,), jnp.float32),
            pltpu.VMEM((_CHUNK * _K,), jnp.float32),
            pltpu.VMEM((_CHUNK * _K,), jnp.int32),
            pltpu.VMEM((_CHUNK * _K,), jnp.int32),
            pltpu.SemaphoreType.DMA,
            pltpu.SemaphoreType.DMA,
            pltpu.SemaphoreType.DMA,
            pltpu.SemaphoreType.DMA,
        ],
    )(probs)
    return vals.reshape(tokens, _K), idx.reshape(tokens, _K)


def kernel(x, W):
    b, n, d = x.shape
    x_flat = x.reshape(b * n, d)
    probs = _tc_probs(x_flat, W)
    vals, idx = _sc_topk(probs)
    return (probs, vals, idx)
